# Initial kernel scaffold; baseline (speedup 1.0000x reference)
import jax
import jax.numpy as jnp
from jax.experimental import pallas as pl


def kernel(x, edge_index, W_l, b_l, W_r, bn_gamma, bn_beta, bn_mean, bn_var):
    raise NotImplementedError("write your pallas kernel here")


# speedup vs baseline: 3.3619x; 3.3619x over previous
import functools

import jax
import jax.numpy as jnp
from jax import lax
from jax.experimental import pallas as pl
from jax.experimental.pallas import tpu as pltpu
from jax.experimental.pallas import tpu_sc as plsc

N = 10000
E = 160000
D = 256
DH = 128
NUM_LAYERS = 3
BN_EPS = 1e-5

NSC = 2
NTILE = 16
CHUNK = 128
NROWS = 10240
ROWS_PT = NROWS // NTILE
EPAD = 161792
EP_TILE = EPAD // NTILE
NCHUNKS = EP_TILE // CHUNK
NB = 400
NBLK = N // NB


def _proj_body(h_ref, wl_ref, wr_ref, b_ref, p_ref, r_ref):
    h = h_ref[...]
    p_ref[...] = jnp.dot(h, wl_ref[...], preferred_element_type=jnp.float32)
    r_ref[...] = jnp.dot(h, wr_ref[...], preferred_element_type=jnp.float32) + b_ref[...]


@jax.jit
def _tc_proj(h, wl, wr, b):
    return pl.pallas_call(
        _proj_body,
        grid=(NBLK, NSC),
        in_specs=[
            pl.BlockSpec((NB, D), lambda i, c: (i, 0)),
            pl.BlockSpec((D, DH), lambda i, c: (0, c)),
            pl.BlockSpec((D, DH), lambda i, c: (0, c)),
            pl.BlockSpec((1, DH), lambda i, c: (0, c)),
        ],
        out_specs=[
            pl.BlockSpec((NB, DH), lambda i, c: (c * NBLK + i, 0)),
            pl.BlockSpec((NB, DH), lambda i, c: (i, c)),
        ],
        out_shape=[
            jax.ShapeDtypeStruct((NSC * N, DH), jnp.float32),
            jax.ShapeDtypeStruct((N, D), jnp.float32),
        ],
    )(h, wl, wr, b)


def _sc_segsum_body(p2, srch, dsth, agg_o, deg_o,
                    srcv, dstv, rows, ones, zrow, z1, acc, dacc, sem):
    c = lax.axis_index("c")
    s = lax.axis_index("s")

    @pl.loop(0, CHUNK, step=16)
    def _(j):
        ones[pl.ds(j, 16)] = jnp.full((16,), 1.0, jnp.float32)
        z1[pl.ds(j, 16)] = jnp.zeros((16,), jnp.float32)

    @pl.loop(0, 128)
    def _(r):
        @pl.loop(0, DH, step=16)
        def _(j):
            zrow[r, pl.ds(j, 16)] = jnp.zeros((16,), jnp.float32)

    @pl.loop(128, ROWS_PT, step=16)
    def _(j):
        z1[pl.ds(j, 16)] = jnp.zeros((16,), jnp.float32)

    base = s * ROWS_PT
    for k in range(ROWS_PT // 128):
        pltpu.sync_copy(zrow, acc.at[pl.ds(base + k * 128, 128)])
    pltpu.sync_copy(z1, dacc.at[pl.ds(base, ROWS_PT)])
    plsc.subcore_barrier()

    ebase = s * EP_TILE
    coff = c * N

    @pl.loop(0, NCHUNKS)
    def _(i):
        off = ebase + i * CHUNK
        pltpu.sync_copy(srch.at[pl.ds(off, CHUNK)], srcv)
        pltpu.sync_copy(dsth.at[pl.ds(off, CHUNK)], dstv)

        @pl.loop(0, CHUNK, step=16)
        def _(j):
            srcv[pl.ds(j, 16)] = srcv[pl.ds(j, 16)] + coff

        pltpu.async_copy(p2.at[srcv], rows, sem).wait()
        pltpu.sync_copy(rows, acc.at[dstv], add=True)

        @pl.when(c == 0)
        def _():
            pltpu.sync_copy(ones, dacc.at[dstv], add=True)

    plsc.subcore_barrier()

    pltpu.sync_copy(acc.at[pl.ds(base, ROWS_PT)],
                    agg_o.at[c, pl.ds(base, ROWS_PT)])

    @pl.when(c == 0)
    def _():
        pltpu.sync_copy(dacc.at[pl.ds(base, ROWS_PT)], deg_o.at[pl.ds(base, ROWS_PT)])


@jax.jit
def _sc_segsum(p2, srcp, dstp):
    mesh = plsc.VectorSubcoreMesh(core_axis_name="c", subcore_axis_name="s")
    k = pl.kernel(
        _sc_segsum_body,
        mesh=mesh,
        out_type=[
            jax.ShapeDtypeStruct((NSC, NROWS, DH), jnp.float32),
            jax.ShapeDtypeStruct((NROWS,), jnp.float32),
        ],
        scratch_types=[
            pltpu.VMEM((CHUNK,), jnp.int32),
            pltpu.VMEM((CHUNK,), jnp.int32),
            pltpu.VMEM((CHUNK, DH), jnp.float32),
            pltpu.VMEM((CHUNK,), jnp.float32),
            pltpu.VMEM((128, DH), jnp.float32),
            pltpu.VMEM((ROWS_PT,), jnp.float32),
            pltpu.VMEM_SHARED((NROWS, DH), jnp.float32),
            pltpu.VMEM_SHARED((NROWS,), jnp.float32),
            pltpu.SemaphoreType.DMA,
        ],
    )
    return k(p2, srcp, dstp)


def _combine_body(bn, agg_ref, r_ref, deg_ref, g_ref, be_ref, m_ref, v_ref, o_ref):
    dinv = 1.0 / jnp.maximum(deg_ref[...], 1.0)
    h = agg_ref[0] * dinv + r_ref[...]
    if bn:
        scale = g_ref[...] * lax.rsqrt(v_ref[...] + BN_EPS)
        h = (h - m_ref[...]) * scale + be_ref[...]
        h = jnp.maximum(h, 0.0)
    o_ref[...] = h


def _make_combine(bn):
    return pl.pallas_call(
        functools.partial(_combine_body, bn),
        grid=(NBLK, NSC),
        in_specs=[
            pl.BlockSpec((1, NB, DH), lambda i, c: (c, i, 0)),
            pl.BlockSpec((NB, DH), lambda i, c: (i, c)),
            pl.BlockSpec((NB, 1), lambda i, c: (i, 0)),
            pl.BlockSpec((1, DH), lambda i, c: (0, c)),
            pl.BlockSpec((1, DH), lambda i, c: (0, c)),
            pl.BlockSpec((1, DH), lambda i, c: (0, c)),
            pl.BlockSpec((1, DH), lambda i, c: (0, c)),
        ],
        out_specs=pl.BlockSpec((NB, DH), lambda i, c: (i, c)),
        out_shape=jax.ShapeDtypeStruct((N, D), jnp.float32),
    )


@jax.jit
def _tc_combine_bn(agg, r, deg, g, be, m, v):
    return _make_combine(True)(agg, r, deg, g, be, m, v)


@jax.jit
def _tc_combine_last(agg, r, deg, g, be, m, v):
    return _make_combine(False)(agg, r, deg, g, be, m, v)


def kernel(x, edge_index, W_l, b_l, W_r, bn_gamma, bn_beta, bn_mean, bn_var):
    pad = EPAD - E
    iota = jnp.arange(pad, dtype=jnp.int32)
    srcp = jnp.concatenate([edge_index[0], iota % N])
    dstp = jnp.concatenate([edge_index[1], N + iota % (NROWS - N)])

    h = x
    deg_col = None
    ident = jnp.zeros((1, D), jnp.float32)
    for i in range(NUM_LAYERS):
        p2, r = _tc_proj(h, W_l[i], W_r[i], b_l[i].reshape(1, D))
        agg, degp = _sc_segsum(p2, srcp, dstp)
        if i == 0:
            deg_col = degp[:N].reshape(N, 1)
        if i < NUM_LAYERS - 1:
            h = _tc_combine_bn(agg, r, deg_col,
                               bn_gamma[i].reshape(1, D), bn_beta[i].reshape(1, D),
                               bn_mean[i].reshape(1, D), bn_var[i].reshape(1, D))
        else:
            h = _tc_combine_last(agg, r, deg_col, ident, ident, ident, ident)
    return h


# speedup vs baseline: 5.8401x; 1.7372x over previous
import functools

import jax
import jax.numpy as jnp
from jax import lax
from jax.experimental import pallas as pl
from jax.experimental.pallas import tpu as pltpu
from jax.experimental.pallas import tpu_sc as plsc

N = 10000
E = 160000
D = 256
DH = 128
NUM_LAYERS = 3
BN_EPS = 1e-5

NSC = 2
NTILE = 16
CHUNK = 128
NROWS = 10240
ROWS_PT = NROWS // NTILE
NCHUNKS = 80
EP_TILE = NCHUNKS * CHUNK
EPAD = NTILE * EP_TILE
NB = 400
NBLK = N // NB


def _proj_body(h_ref, wl_ref, wr_ref, b_ref, p_ref, r_ref):
    h = h_ref[...]
    p_ref[...] = jnp.dot(h, wl_ref[...], preferred_element_type=jnp.float32)
    r_ref[...] = jnp.dot(h, wr_ref[...], preferred_element_type=jnp.float32) + b_ref[...]


@jax.jit
def _tc_proj(h, wl, wr, b):
    return pl.pallas_call(
        _proj_body,
        grid=(NBLK, NSC),
        in_specs=[
            pl.BlockSpec((NB, D), lambda i, c: (i, 0)),
            pl.BlockSpec((D, DH), lambda i, c: (0, c)),
            pl.BlockSpec((D, DH), lambda i, c: (0, c)),
            pl.BlockSpec((1, DH), lambda i, c: (0, c)),
        ],
        out_specs=[
            pl.BlockSpec((NB, DH), lambda i, c: (c * NBLK + i, 0)),
            pl.BlockSpec((NB, DH), lambda i, c: (i, c)),
        ],
        out_shape=[
            jax.ShapeDtypeStruct((NSC * N, DH), jnp.float32),
            jax.ShapeDtypeStruct((N, D), jnp.float32),
        ],
    )(h, wl, wr, b)


def _sc_segsum_body(deg, p2, srch, dsth, agg_o, deg_o,
                    srcv, dstv, rows0, rows1, ones, z1,
                    acc, dacc,
                    semi0, semi1, semi2, semi3, semg0, semg1):
    c = lax.axis_index("c")
    s = lax.axis_index("s")
    rows = (rows0, rows1)
    semi = (semi0, semi1, semi2, semi3)
    semg = (semg0, semg1)
    row0 = s * NCHUNKS

    def idx_start(k, q):
        pltpu.async_copy(srch.at[c, row0 + k], srcv.at[q], semi[q])
        pltpu.async_copy(dsth.at[row0 + k], dstv.at[q], semi[q])

    def idx_wait(k, q):
        pltpu.make_async_copy(srch.at[c, row0 + k], srcv.at[q], semi[q]).wait()
        pltpu.make_async_copy(dsth.at[row0 + k], dstv.at[q], semi[q]).wait()

    def gat_start(k, b, q):
        pltpu.async_copy(p2.at[srcv.at[q]], rows[b], semg[b])

    def gat_wait(b):
        pltpu.make_async_copy(p2.at[srcv.at[0]], rows[b], semg[b]).wait()

    for q in range(4):
        idx_start(q, q)

    if deg:
        @pl.loop(0, CHUNK, step=16)
        def _(j):
            ones[pl.ds(j, 16)] = jnp.full((16,), 1.0, jnp.float32)

        @pl.loop(0, ROWS_PT, step=16)
        def _(j):
            z1[pl.ds(j, 16)] = jnp.zeros((16,), jnp.float32)

    @pl.loop(0, CHUNK)
    def _(r):
        @pl.loop(0, DH, step=16)
        def _(j):
            rows0[r, pl.ds(j, 16)] = jnp.zeros((16,), jnp.float32)

    base = s * ROWS_PT
    for k in range(ROWS_PT // CHUNK):
        pltpu.sync_copy(rows0, acc.at[pl.ds(base + k * CHUNK, CHUNK)])
    if deg:
        pltpu.sync_copy(z1, dacc.at[pl.ds(base, ROWS_PT)])
    plsc.subcore_barrier()

    idx_wait(0, 0)
    gat_start(0, 0, 0)
    idx_wait(1, 1)
    gat_start(1, 1, 1)

    @pl.loop(0, NCHUNKS, step=4)
    def _(i):
        for u in range(4):
            k = i + u
            b = u % 2
            q = u
            gat_wait(b)
            pltpu.sync_copy(rows[b], acc.at[dstv.at[q]], add=True)
            if deg:
                @pl.when(c == 0)
                def _():
                    pltpu.sync_copy(ones, dacc.at[dstv.at[q]], add=True)

            @pl.when(k + 4 < NCHUNKS)
            def _():
                idx_start(k + 4, q)

            @pl.when(k + 2 < NCHUNKS)
            def _():
                idx_wait(k + 2, (u + 2) % 4)
                gat_start(k + 2, b, (u + 2) % 4)

    plsc.subcore_barrier()

    pltpu.sync_copy(acc.at[pl.ds(base, ROWS_PT)],
                    agg_o.at[c, pl.ds(base, ROWS_PT)])

    if deg:
        @pl.when(c == 0)
        def _():
            pltpu.sync_copy(dacc.at[pl.ds(base, ROWS_PT)],
                            deg_o.at[pl.ds(base, ROWS_PT)])


def _make_sc_segsum(deg):
    mesh = plsc.VectorSubcoreMesh(core_axis_name="c", subcore_axis_name="s")
    return pl.kernel(
        functools.partial(_sc_segsum_body, deg),
        mesh=mesh,
        out_type=[
            jax.ShapeDtypeStruct((NSC, NROWS, DH), jnp.float32),
            jax.ShapeDtypeStruct((NROWS,), jnp.float32),
        ],
        scratch_types=[
            pltpu.VMEM((4, CHUNK), jnp.int32),
            pltpu.VMEM((4, CHUNK), jnp.int32),
            pltpu.VMEM((CHUNK, DH), jnp.float32),
            pltpu.VMEM((CHUNK, DH), jnp.float32),
            pltpu.VMEM((CHUNK,), jnp.float32),
            pltpu.VMEM((ROWS_PT,), jnp.float32),
            pltpu.VMEM_SHARED((NROWS, DH), jnp.float32),
            pltpu.VMEM_SHARED((NROWS,), jnp.float32),
            pltpu.SemaphoreType.DMA,
            pltpu.SemaphoreType.DMA,
            pltpu.SemaphoreType.DMA,
            pltpu.SemaphoreType.DMA,
            pltpu.SemaphoreType.DMA,
            pltpu.SemaphoreType.DMA,
        ],
    )


@jax.jit
def _sc_segsum_deg(p2, srcp, dstp):
    return _make_sc_segsum(True)(p2, srcp, dstp)


@jax.jit
def _sc_segsum(p2, srcp, dstp):
    return _make_sc_segsum(False)(p2, srcp, dstp)


def _combine_body(bn, agg_ref, r_ref, deg_ref, g_ref, be_ref, m_ref, v_ref, o_ref):
    dinv = 1.0 / jnp.maximum(deg_ref[...], 1.0)
    h = agg_ref[0] * dinv + r_ref[...]
    if bn:
        scale = g_ref[...] * lax.rsqrt(v_ref[...] + BN_EPS)
        h = (h - m_ref[...]) * scale + be_ref[...]
        h = jnp.maximum(h, 0.0)
    o_ref[...] = h


def _make_combine(bn):
    return pl.pallas_call(
        functools.partial(_combine_body, bn),
        grid=(NBLK, NSC),
        in_specs=[
            pl.BlockSpec((1, NB, DH), lambda i, c: (c, i, 0)),
            pl.BlockSpec((NB, DH), lambda i, c: (i, c)),
            pl.BlockSpec((NB, 1), lambda i, c: (i, 0)),
            pl.BlockSpec((1, DH), lambda i, c: (0, c)),
            pl.BlockSpec((1, DH), lambda i, c: (0, c)),
            pl.BlockSpec((1, DH), lambda i, c: (0, c)),
            pl.BlockSpec((1, DH), lambda i, c: (0, c)),
        ],
        out_specs=pl.BlockSpec((NB, DH), lambda i, c: (i, c)),
        out_shape=jax.ShapeDtypeStruct((N, D), jnp.float32),
    )


@jax.jit
def _tc_combine_bn(agg, r, deg, g, be, m, v):
    return _make_combine(True)(agg, r, deg, g, be, m, v)


@jax.jit
def _tc_combine_last(agg, r, deg, g, be, m, v):
    return _make_combine(False)(agg, r, deg, g, be, m, v)


def kernel(x, edge_index, W_l, b_l, W_r, bn_gamma, bn_beta, bn_mean, bn_var):
    pad = EPAD - E
    iota = jnp.arange(pad, dtype=jnp.int32)
    src = jnp.concatenate([edge_index[0], iota % N])
    srcp = jnp.stack([src, src + N]).reshape(NSC, NTILE * NCHUNKS, CHUNK)
    dstp = jnp.concatenate([edge_index[1], N + iota % (NROWS - N)])
    dstp = dstp.reshape(NTILE * NCHUNKS, CHUNK)

    h = x
    deg_col = None
    ident = jnp.zeros((1, D), jnp.float32)
    for i in range(NUM_LAYERS):
        p2, r = _tc_proj(h, W_l[i], W_r[i], b_l[i].reshape(1, D))
        if i == 0:
            agg, degp = _sc_segsum_deg(p2, srcp, dstp)
            deg_col = degp[:N].reshape(N, 1)
        else:
            agg, _ = _sc_segsum(p2, srcp, dstp)
        if i < NUM_LAYERS - 1:
            h = _tc_combine_bn(agg, r, deg_col,
                               bn_gamma[i].reshape(1, D), bn_beta[i].reshape(1, D),
                               bn_mean[i].reshape(1, D), bn_var[i].reshape(1, D))
        else:
            h = _tc_combine_last(agg, r, deg_col, ident, ident, ident, ident)
    return h


# speedup vs baseline: 7.8264x; 1.3401x over previous
import functools

import jax
import jax.numpy as jnp
from jax import lax
from jax.experimental import pallas as pl
from jax.experimental.pallas import tpu as pltpu
from jax.experimental.pallas import tpu_sc as plsc

N = 10000
E = 160000
D = 256
DH = 128
NUM_LAYERS = 3
BN_EPS = 1e-5

NSC = 2
NTILE = 16
CHUNK = 128
NROWS = 10240
ROWS_PT = NROWS // NTILE
NCHUNKS = 80
EP_TILE = NCHUNKS * CHUNK
EPAD = NTILE * EP_TILE
NB = 2000
NBLK = N // NB


def _proj_body(h_ref, wl_ref, wr_ref, b_ref, p_ref, r_ref):
    h = h_ref[...]
    p_ref[...] = jnp.dot(h, wl_ref[...], preferred_element_type=jnp.float32)
    r_ref[...] = jnp.dot(h, wr_ref[...], preferred_element_type=jnp.float32) + b_ref[...]


@jax.jit
def _tc_proj(h, wl, wr, b):
    return pl.pallas_call(
        _proj_body,
        grid=(NBLK, NSC),
        in_specs=[
            pl.BlockSpec((NB, D), lambda i, c: (i, 0)),
            pl.BlockSpec((D, DH), lambda i, c: (0, c)),
            pl.BlockSpec((D, DH), lambda i, c: (0, c)),
            pl.BlockSpec((1, DH), lambda i, c: (0, c)),
        ],
        out_specs=[
            pl.BlockSpec((NB, DH), lambda i, c: (c * NBLK + i, 0)),
            pl.BlockSpec((NB, DH), lambda i, c: (i, c)),
        ],
        out_shape=[
            jax.ShapeDtypeStruct((NSC * N, DH), jnp.float32),
            jax.ShapeDtypeStruct((N, D), jnp.float32),
        ],
    )(h, wl, wr, b)


def _fused_body(agg_ref, r_ref, deg_ref, g_ref, be_ref, m_ref, v_ref,
                wl_ref, wr_ref, b_ref, p_ref, r2_ref):
    dinv = 1.0 / jnp.maximum(deg_ref[...], 1.0)
    h = jnp.concatenate([agg_ref[0], agg_ref[1]], axis=1) * dinv + r_ref[...]
    scale = g_ref[...] * lax.rsqrt(v_ref[...] + BN_EPS)
    h = jnp.maximum((h - m_ref[...]) * scale + be_ref[...], 0.0)
    p_ref[...] = jnp.dot(h, wl_ref[...], preferred_element_type=jnp.float32)
    r2_ref[...] = jnp.dot(h, wr_ref[...], preferred_element_type=jnp.float32) + b_ref[...]


@jax.jit
def _tc_fused(agg, r, deg, g, be, m, v, wl, wr, b):
    return pl.pallas_call(
        _fused_body,
        grid=(NBLK, NSC),
        in_specs=[
            pl.BlockSpec((NSC, NB, DH), lambda i, c: (0, i, 0)),
            pl.BlockSpec((NB, D), lambda i, c: (i, 0)),
            pl.BlockSpec((NB, 1), lambda i, c: (i, 0)),
            pl.BlockSpec((1, D), lambda i, c: (0, 0)),
            pl.BlockSpec((1, D), lambda i, c: (0, 0)),
            pl.BlockSpec((1, D), lambda i, c: (0, 0)),
            pl.BlockSpec((1, D), lambda i, c: (0, 0)),
            pl.BlockSpec((D, DH), lambda i, c: (0, c)),
            pl.BlockSpec((D, DH), lambda i, c: (0, c)),
            pl.BlockSpec((1, DH), lambda i, c: (0, c)),
        ],
        out_specs=[
            pl.BlockSpec((NB, DH), lambda i, c: (c * NBLK + i, 0)),
            pl.BlockSpec((NB, DH), lambda i, c: (i, c)),
        ],
        out_shape=[
            jax.ShapeDtypeStruct((NSC * N, DH), jnp.float32),
            jax.ShapeDtypeStruct((N, D), jnp.float32),
        ],
    )(agg, r, deg, g, be, m, v, wl, wr, b)


def _sc_segsum_body(deg, p2, srch, dsth, agg_o, deg_o,
                    srcv, dstv, rows0, rows1, ones, z1,
                    acc, dacc,
                    semi0, semi1, semi2, semi3, semg0, semg1):
    c = lax.axis_index("c")
    s = lax.axis_index("s")
    rows = (rows0, rows1)
    semi = (semi0, semi1, semi2, semi3)
    semg = (semg0, semg1)
    row0 = s * NCHUNKS

    def idx_start(k, q):
        pltpu.async_copy(srch.at[c, row0 + k], srcv.at[q], semi[q])
        pltpu.async_copy(dsth.at[row0 + k], dstv.at[q], semi[q])

    def idx_wait(k, q):
        pltpu.make_async_copy(srch.at[c, row0 + k], srcv.at[q], semi[q]).wait()
        pltpu.make_async_copy(dsth.at[row0 + k], dstv.at[q], semi[q]).wait()

    def gat_start(k, b, q):
        pltpu.async_copy(p2.at[srcv.at[q]], rows[b], semg[b])

    def gat_wait(b):
        pltpu.make_async_copy(p2.at[srcv.at[0]], rows[b], semg[b]).wait()

    for q in range(4):
        idx_start(q, q)

    if deg:
        @pl.loop(0, CHUNK, step=16)
        def _(j):
            ones[pl.ds(j, 16)] = jnp.full((16,), 1.0, jnp.float32)

        @pl.loop(0, ROWS_PT, step=16)
        def _(j):
            z1[pl.ds(j, 16)] = jnp.zeros((16,), jnp.float32)

    @pl.loop(0, CHUNK)
    def _(r):
        @pl.loop(0, DH, step=16)
        def _(j):
            rows0[r, pl.ds(j, 16)] = jnp.zeros((16,), jnp.float32)

    base = s * ROWS_PT
    for k in range(ROWS_PT // CHUNK):
        pltpu.sync_copy(rows0, acc.at[pl.ds(base + k * CHUNK, CHUNK)])
    if deg:
        pltpu.sync_copy(z1, dacc.at[pl.ds(base, ROWS_PT)])
    plsc.subcore_barrier()

    idx_wait(0, 0)
    gat_start(0, 0, 0)
    idx_wait(1, 1)
    gat_start(1, 1, 1)

    @pl.loop(0, NCHUNKS, step=4)
    def _(i):
        for u in range(4):
            k = i + u
            b = u % 2
            q = u
            gat_wait(b)
            pltpu.sync_copy(rows[b], acc.at[dstv.at[q]], add=True)
            if deg:
                @pl.when(c == 0)
                def _():
                    pltpu.sync_copy(ones, dacc.at[dstv.at[q]], add=True)

            @pl.when(k + 4 < NCHUNKS)
            def _():
                idx_start(k + 4, q)

            @pl.when(k + 2 < NCHUNKS)
            def _():
                idx_wait(k + 2, (u + 2) % 4)
                gat_start(k + 2, b, (u + 2) % 4)

    plsc.subcore_barrier()

    pltpu.sync_copy(acc.at[pl.ds(base, ROWS_PT)],
                    agg_o.at[c, pl.ds(base, ROWS_PT)])

    if deg:
        @pl.when(c == 0)
        def _():
            pltpu.sync_copy(dacc.at[pl.ds(base, ROWS_PT)],
                            deg_o.at[pl.ds(base, ROWS_PT)])


def _make_sc_segsum(deg):
    mesh = plsc.VectorSubcoreMesh(core_axis_name="c", subcore_axis_name="s")
    return pl.kernel(
        functools.partial(_sc_segsum_body, deg),
        mesh=mesh,
        out_type=[
            jax.ShapeDtypeStruct((NSC, NROWS, DH), jnp.float32),
            jax.ShapeDtypeStruct((NROWS,), jnp.float32),
        ],
        scratch_types=[
            pltpu.VMEM((4, CHUNK), jnp.int32),
            pltpu.VMEM((4, CHUNK), jnp.int32),
            pltpu.VMEM((CHUNK, DH), jnp.float32),
            pltpu.VMEM((CHUNK, DH), jnp.float32),
            pltpu.VMEM((CHUNK,), jnp.float32),
            pltpu.VMEM((ROWS_PT,), jnp.float32),
            pltpu.VMEM_SHARED((NROWS, DH), jnp.float32),
            pltpu.VMEM_SHARED((NROWS,), jnp.float32),
            pltpu.SemaphoreType.DMA,
            pltpu.SemaphoreType.DMA,
            pltpu.SemaphoreType.DMA,
            pltpu.SemaphoreType.DMA,
            pltpu.SemaphoreType.DMA,
            pltpu.SemaphoreType.DMA,
        ],
    )


@jax.jit
def _sc_segsum_deg(p2, srcp, dstp):
    return _make_sc_segsum(True)(p2, srcp, dstp)


@jax.jit
def _sc_segsum(p2, srcp, dstp):
    return _make_sc_segsum(False)(p2, srcp, dstp)


def _last_body(agg_ref, r_ref, deg_ref, o_ref):
    dinv = 1.0 / jnp.maximum(deg_ref[...], 1.0)
    o_ref[...] = agg_ref[0] * dinv + r_ref[...]


@jax.jit
def _tc_combine_last(agg, r, deg):
    return pl.pallas_call(
        _last_body,
        grid=(NBLK, NSC),
        in_specs=[
            pl.BlockSpec((1, NB, DH), lambda i, c: (c, i, 0)),
            pl.BlockSpec((NB, DH), lambda i, c: (i, c)),
            pl.BlockSpec((NB, 1), lambda i, c: (i, 0)),
        ],
        out_specs=pl.BlockSpec((NB, DH), lambda i, c: (i, c)),
        out_shape=jax.ShapeDtypeStruct((N, D), jnp.float32),
    )(agg, r, deg)


def kernel(x, edge_index, W_l, b_l, W_r, bn_gamma, bn_beta, bn_mean, bn_var):
    pad = EPAD - E
    iota = jnp.arange(pad, dtype=jnp.int32)
    src = jnp.concatenate([edge_index[0], iota % N])
    srcp = jnp.stack([src, src + N]).reshape(NSC, NTILE * NCHUNKS, CHUNK)
    dstp = jnp.concatenate([edge_index[1], N + iota % (NROWS - N)])
    dstp = dstp.reshape(NTILE * NCHUNKS, CHUNK)

    p2, r = _tc_proj(x, W_l[0], W_r[0], b_l[0].reshape(1, D))
    agg, degp = _sc_segsum_deg(p2, srcp, dstp)
    deg_col = degp[:N].reshape(N, 1)
    for i in range(1, NUM_LAYERS):
        p2, r = _tc_fused(agg, r, deg_col,
                          bn_gamma[i - 1].reshape(1, D), bn_beta[i - 1].reshape(1, D),
                          bn_mean[i - 1].reshape(1, D), bn_var[i - 1].reshape(1, D),
                          W_l[i], W_r[i], b_l[i].reshape(1, D))
        agg, _ = _sc_segsum(p2, srcp, dstp)
    return _tc_combine_last(agg, r, deg_col)


# speedup vs baseline: 7.8493x; 1.0029x over previous
import functools

import jax
import jax.numpy as jnp
from jax import lax
from jax.experimental import pallas as pl
from jax.experimental.pallas import tpu as pltpu
from jax.experimental.pallas import tpu_sc as plsc

N = 10000
E = 160000
D = 256
DH = 128
NUM_LAYERS = 3
BN_EPS = 1e-5

NSC = 2
NTILE = 16
CHUNK = 128
NROWS = 10240
ROWS_PT = NROWS // NTILE
NCHUNKS = 80
EP_TILE = NCHUNKS * CHUNK
EPAD = NTILE * EP_TILE
NB = 2000
NBLK = N // NB


def _proj_body(h_ref, w_ref, b_ref, p_ref, r_ref):
    h = h_ref[...].astype(jnp.bfloat16)
    out = jnp.dot(h, w_ref[0].astype(jnp.bfloat16),
                  preferred_element_type=jnp.float32)
    p_ref[...] = out[:, :DH]
    r_ref[...] = out[:, DH:] + b_ref[...]


@jax.jit
def _tc_proj(h, wcat, b):
    return pl.pallas_call(
        _proj_body,
        grid=(NBLK, NSC),
        in_specs=[
            pl.BlockSpec((NB, D), lambda i, c: (i, 0)),
            pl.BlockSpec((1, D, D), lambda i, c: (c, 0, 0)),
            pl.BlockSpec((1, DH), lambda i, c: (0, c)),
        ],
        out_specs=[
            pl.BlockSpec((NB, DH), lambda i, c: (c * NBLK + i, 0)),
            pl.BlockSpec((NB, DH), lambda i, c: (i, c)),
        ],
        out_shape=[
            jax.ShapeDtypeStruct((NSC * N, DH), jnp.float32),
            jax.ShapeDtypeStruct((N, D), jnp.float32),
        ],
    )(h, wcat, b)


def _fused_body(agg_ref, r_ref, deg_ref, g_ref, be_ref, m_ref, v_ref,
                w_ref, b_ref, p_ref, r2_ref):
    dinv = 1.0 / jnp.maximum(deg_ref[...], 1.0)
    h = jnp.concatenate([agg_ref[0], agg_ref[1]], axis=1) * dinv + r_ref[...]
    scale = g_ref[...] * lax.rsqrt(v_ref[...] + BN_EPS)
    h = jnp.maximum((h - m_ref[...]) * scale + be_ref[...], 0.0)
    out = jnp.dot(h.astype(jnp.bfloat16), w_ref[0].astype(jnp.bfloat16),
                  preferred_element_type=jnp.float32)
    p_ref[...] = out[:, :DH]
    r2_ref[...] = out[:, DH:] + b_ref[...]


@jax.jit
def _tc_fused(agg, r, deg, g, be, m, v, wcat, b):
    return pl.pallas_call(
        _fused_body,
        grid=(NBLK, NSC),
        in_specs=[
            pl.BlockSpec((NSC, NB, DH), lambda i, c: (0, i, 0)),
            pl.BlockSpec((NB, D), lambda i, c: (i, 0)),
            pl.BlockSpec((NB, 1), lambda i, c: (i, 0)),
            pl.BlockSpec((1, D), lambda i, c: (0, 0)),
            pl.BlockSpec((1, D), lambda i, c: (0, 0)),
            pl.BlockSpec((1, D), lambda i, c: (0, 0)),
            pl.BlockSpec((1, D), lambda i, c: (0, 0)),
            pl.BlockSpec((1, D, D), lambda i, c: (c, 0, 0)),
            pl.BlockSpec((1, DH), lambda i, c: (0, c)),
        ],
        out_specs=[
            pl.BlockSpec((NB, DH), lambda i, c: (c * NBLK + i, 0)),
            pl.BlockSpec((NB, DH), lambda i, c: (i, c)),
        ],
        out_shape=[
            jax.ShapeDtypeStruct((NSC * N, DH), jnp.float32),
            jax.ShapeDtypeStruct((N, D), jnp.float32),
        ],
    )(agg, r, deg, g, be, m, v, wcat, b)


def _sc_segsum_body(deg, p2, srch, dsth, agg_o, deg_o,
                    srcv, dstv, rows0, rows1, ones, z1,
                    acc, dacc,
                    semi0, semi1, semi2, semi3, semg0, semg1):
    c = lax.axis_index("c")
    s = lax.axis_index("s")
    rows = (rows0, rows1)
    semi = (semi0, semi1, semi2, semi3)
    semg = (semg0, semg1)
    row0 = s * NCHUNKS

    def idx_start(k, q):
        pltpu.async_copy(srch.at[c, row0 + k], srcv.at[q], semi[q])
        pltpu.async_copy(dsth.at[row0 + k], dstv.at[q], semi[q])

    def idx_wait(k, q):
        pltpu.make_async_copy(srch.at[c, row0 + k], srcv.at[q], semi[q]).wait()
        pltpu.make_async_copy(dsth.at[row0 + k], dstv.at[q], semi[q]).wait()

    def gat_start(k, b, q):
        pltpu.async_copy(p2.at[srcv.at[q]], rows[b], semg[b])

    def gat_wait(b):
        pltpu.make_async_copy(p2.at[srcv.at[0]], rows[b], semg[b]).wait()

    for q in range(4):
        idx_start(q, q)

    if deg:
        @pl.loop(0, CHUNK, step=16)
        def _(j):
            ones[pl.ds(j, 16)] = jnp.full((16,), 1.0, jnp.float32)

        @pl.loop(0, ROWS_PT, step=16)
        def _(j):
            z1[pl.ds(j, 16)] = jnp.zeros((16,), jnp.float32)

    @pl.loop(0, CHUNK)
    def _(r):
        @pl.loop(0, DH, step=16)
        def _(j):
            rows0[r, pl.ds(j, 16)] = jnp.zeros((16,), jnp.float32)

    base = s * ROWS_PT
    for k in range(ROWS_PT // CHUNK):
        pltpu.sync_copy(rows0, acc.at[pl.ds(base + k * CHUNK, CHUNK)])
    if deg:
        pltpu.sync_copy(z1, dacc.at[pl.ds(base, ROWS_PT)])
    plsc.subcore_barrier()

    idx_wait(0, 0)
    gat_start(0, 0, 0)
    idx_wait(1, 1)
    gat_start(1, 1, 1)

    @pl.loop(0, NCHUNKS, step=4)
    def _(i):
        for u in range(4):
            k = i + u
            b = u % 2
            q = u
            gat_wait(b)
            pltpu.sync_copy(rows[b], acc.at[dstv.at[q]], add=True)
            if deg:
                @pl.when(c == 0)
                def _():
                    pltpu.sync_copy(ones, dacc.at[dstv.at[q]], add=True)

            @pl.when(k + 4 < NCHUNKS)
            def _():
                idx_start(k + 4, q)

            @pl.when(k + 2 < NCHUNKS)
            def _():
                idx_wait(k + 2, (u + 2) % 4)
                gat_start(k + 2, b, (u + 2) % 4)

    plsc.subcore_barrier()

    pltpu.sync_copy(acc.at[pl.ds(base, ROWS_PT)],
                    agg_o.at[c, pl.ds(base, ROWS_PT)])

    if deg:
        @pl.when(c == 0)
        def _():
            pltpu.sync_copy(dacc.at[pl.ds(base, ROWS_PT)],
                            deg_o.at[pl.ds(base, ROWS_PT)])


def _make_sc_segsum(deg):
    mesh = plsc.VectorSubcoreMesh(core_axis_name="c", subcore_axis_name="s")
    return pl.kernel(
        functools.partial(_sc_segsum_body, deg),
        mesh=mesh,
        out_type=[
            jax.ShapeDtypeStruct((NSC, NROWS, DH), jnp.float32),
            jax.ShapeDtypeStruct((NROWS,), jnp.float32),
        ],
        scratch_types=[
            pltpu.VMEM((4, CHUNK), jnp.int32),
            pltpu.VMEM((4, CHUNK), jnp.int32),
            pltpu.VMEM((CHUNK, DH), jnp.float32),
            pltpu.VMEM((CHUNK, DH), jnp.float32),
            pltpu.VMEM((CHUNK,), jnp.float32),
            pltpu.VMEM((ROWS_PT,), jnp.float32),
            pltpu.VMEM_SHARED((NROWS, DH), jnp.float32),
            pltpu.VMEM_SHARED((NROWS,), jnp.float32),
            pltpu.SemaphoreType.DMA,
            pltpu.SemaphoreType.DMA,
            pltpu.SemaphoreType.DMA,
            pltpu.SemaphoreType.DMA,
            pltpu.SemaphoreType.DMA,
            pltpu.SemaphoreType.DMA,
        ],
    )


@jax.jit
def _sc_segsum_deg(p2, srcp, dstp):
    return _make_sc_segsum(True)(p2, srcp, dstp)


@jax.jit
def _sc_segsum(p2, srcp, dstp):
    return _make_sc_segsum(False)(p2, srcp, dstp)


def _last_body(agg_ref, r_ref, deg_ref, o_ref):
    dinv = 1.0 / jnp.maximum(deg_ref[...], 1.0)
    o_ref[...] = agg_ref[0] * dinv + r_ref[...]


@jax.jit
def _tc_combine_last(agg, r, deg):
    return pl.pallas_call(
        _last_body,
        grid=(NBLK, NSC),
        in_specs=[
            pl.BlockSpec((1, NB, DH), lambda i, c: (c, i, 0)),
            pl.BlockSpec((NB, DH), lambda i, c: (i, c)),
            pl.BlockSpec((NB, 1), lambda i, c: (i, 0)),
        ],
        out_specs=pl.BlockSpec((NB, DH), lambda i, c: (i, c)),
        out_shape=jax.ShapeDtypeStruct((N, D), jnp.float32),
    )(agg, r, deg)


def kernel(x, edge_index, W_l, b_l, W_r, bn_gamma, bn_beta, bn_mean, bn_var):
    pad = EPAD - E
    iota = jnp.arange(pad, dtype=jnp.int32)
    src = jnp.concatenate([edge_index[0], iota % N])
    srcp = jnp.stack([src, src + N]).reshape(NSC, NTILE * NCHUNKS, CHUNK)
    dstp = jnp.concatenate([edge_index[1], N + iota % (NROWS - N)])
    dstp = dstp.reshape(NTILE * NCHUNKS, CHUNK)

    wl_r = W_l.reshape(NUM_LAYERS, D, NSC, DH).transpose(0, 2, 1, 3)
    wr_r = W_r.reshape(NUM_LAYERS, D, NSC, DH).transpose(0, 2, 1, 3)
    wcat = jnp.concatenate([wl_r, wr_r], axis=3)

    p2, r = _tc_proj(x, wcat[0], b_l[0].reshape(1, D))
    agg, degp = _sc_segsum_deg(p2, srcp, dstp)
    deg_col = degp[:N].reshape(N, 1)
    for i in range(1, NUM_LAYERS):
        p2, r = _tc_fused(agg, r, deg_col,
                          bn_gamma[i - 1].reshape(1, D), bn_beta[i - 1].reshape(1, D),
                          bn_mean[i - 1].reshape(1, D), bn_var[i - 1].reshape(1, D),
                          wcat[i], b_l[i].reshape(1, D))
        agg, _ = _sc_segsum(p2, srcp, dstp)
    return _tc_combine_last(agg, r, deg_col)


# speedup vs baseline: 7.8654x; 1.0020x over previous
import functools

import jax
import jax.numpy as jnp
from jax import lax
from jax.experimental import pallas as pl
from jax.experimental.pallas import tpu as pltpu
from jax.experimental.pallas import tpu_sc as plsc

N = 10000
E = 160000
D = 256
DH = 128
NUM_LAYERS = 3
BN_EPS = 1e-5

NSC = 2
NTILE = 16
CHUNK = 128
NROWS = 10240
ROWS_PT = NROWS // NTILE
NCHUNKS = 80
EP_TILE = NCHUNKS * CHUNK
EPAD = NTILE * EP_TILE
NB = 2000
NBLK = N // NB


def _proj_body(h_ref, w_ref, b_ref, p_ref, r_ref):
    h = h_ref[...].astype(jnp.bfloat16)
    out = jnp.dot(h, w_ref[0].astype(jnp.bfloat16),
                  preferred_element_type=jnp.float32)
    p_ref[...] = out[:, :DH]
    r_ref[...] = out[:, DH:] + b_ref[...]


@jax.jit
def _tc_proj(h, wcat, b):
    return pl.pallas_call(
        _proj_body,
        grid=(NBLK, NSC),
        in_specs=[
            pl.BlockSpec((NB, D), lambda i, c: (i, 0)),
            pl.BlockSpec((1, D, D), lambda i, c: (c, 0, 0)),
            pl.BlockSpec((1, DH), lambda i, c: (0, c)),
        ],
        out_specs=[
            pl.BlockSpec((NB, DH), lambda i, c: (c * NBLK + i, 0)),
            pl.BlockSpec((NB, DH), lambda i, c: (i, c)),
        ],
        out_shape=[
            jax.ShapeDtypeStruct((NSC * N, DH), jnp.float32),
            jax.ShapeDtypeStruct((N, D), jnp.float32),
        ],
    )(h, wcat, b)


def _fused_body(agg_ref, r_ref, deg_ref, g_ref, be_ref, m_ref, v_ref,
                w_ref, b_ref, p_ref, r2_ref):
    dinv = 1.0 / jnp.maximum(deg_ref[...], 1.0)
    h = jnp.concatenate([agg_ref[0], agg_ref[1]], axis=1) * dinv + r_ref[...]
    scale = g_ref[...] * lax.rsqrt(v_ref[...] + BN_EPS)
    h = jnp.maximum((h - m_ref[...]) * scale + be_ref[...], 0.0)
    out = jnp.dot(h.astype(jnp.bfloat16), w_ref[0].astype(jnp.bfloat16),
                  preferred_element_type=jnp.float32)
    p_ref[...] = out[:, :DH]
    r2_ref[...] = out[:, DH:] + b_ref[...]


@jax.jit
def _tc_fused(agg, r, deg, g, be, m, v, wcat, b):
    return pl.pallas_call(
        _fused_body,
        grid=(NBLK, NSC),
        in_specs=[
            pl.BlockSpec((NSC, NB, DH), lambda i, c: (0, i, 0)),
            pl.BlockSpec((NB, D), lambda i, c: (i, 0)),
            pl.BlockSpec((NB, 1), lambda i, c: (i, 0)),
            pl.BlockSpec((1, D), lambda i, c: (0, 0)),
            pl.BlockSpec((1, D), lambda i, c: (0, 0)),
            pl.BlockSpec((1, D), lambda i, c: (0, 0)),
            pl.BlockSpec((1, D), lambda i, c: (0, 0)),
            pl.BlockSpec((1, D, D), lambda i, c: (c, 0, 0)),
            pl.BlockSpec((1, DH), lambda i, c: (0, c)),
        ],
        out_specs=[
            pl.BlockSpec((NB, DH), lambda i, c: (c * NBLK + i, 0)),
            pl.BlockSpec((NB, DH), lambda i, c: (i, c)),
        ],
        out_shape=[
            jax.ShapeDtypeStruct((NSC * N, DH), jnp.float32),
            jax.ShapeDtypeStruct((N, D), jnp.float32),
        ],
    )(agg, r, deg, g, be, m, v, wcat, b)


def _sc_segsum_body(deg, p2, idxh, agg_o, deg_o,
                    idxv, rows0, rows1, ones, z1,
                    acc, dacc,
                    semi0, semi1, semi2, semi3, semg0, semg1):
    c = lax.axis_index("c")
    s = lax.axis_index("s")
    rows = (rows0, rows1)
    semi = (semi0, semi1, semi2, semi3)
    semg = (semg0, semg1)
    row0 = s * NCHUNKS

    def idx_start(k, q):
        pltpu.async_copy(idxh.at[c, row0 + k], idxv.at[q], semi[q])

    def idx_wait(k, q):
        pltpu.make_async_copy(idxh.at[c, row0 + k], idxv.at[q], semi[q]).wait()

    def gat_start(k, b, q):
        pltpu.async_copy(p2.at[idxv.at[q, 0]], rows[b], semg[b])

    def gat_wait(b):
        pltpu.make_async_copy(p2.at[idxv.at[0, 0]], rows[b], semg[b]).wait()

    for q in range(4):
        idx_start(q, q)

    if deg:
        @pl.loop(0, CHUNK, step=16)
        def _(j):
            ones[pl.ds(j, 16)] = jnp.full((16,), 1.0, jnp.float32)

        @pl.loop(0, ROWS_PT, step=16)
        def _(j):
            z1[pl.ds(j, 16)] = jnp.zeros((16,), jnp.float32)

    @pl.loop(0, CHUNK)
    def _(r):
        @pl.loop(0, DH, step=16)
        def _(j):
            rows0[r, pl.ds(j, 16)] = jnp.zeros((16,), jnp.float32)

    base = s * ROWS_PT
    for k in range(ROWS_PT // CHUNK):
        pltpu.sync_copy(rows0, acc.at[pl.ds(base + k * CHUNK, CHUNK)])
    if deg:
        pltpu.sync_copy(z1, dacc.at[pl.ds(base, ROWS_PT)])
    plsc.subcore_barrier()

    idx_wait(0, 0)
    gat_start(0, 0, 0)
    idx_wait(1, 1)
    gat_start(1, 1, 1)

    @pl.loop(0, NCHUNKS, step=4)
    def _(i):
        for u in range(4):
            k = i + u
            b = u % 2
            q = u
            gat_wait(b)
            pltpu.sync_copy(rows[b], acc.at[idxv.at[q, 1]], add=True)
            if deg:
                @pl.when(c == 0)
                def _():
                    pltpu.sync_copy(ones, dacc.at[idxv.at[q, 1]], add=True)

            @pl.when(k + 4 < NCHUNKS)
            def _():
                idx_start(k + 4, q)

            @pl.when(k + 2 < NCHUNKS)
            def _():
                idx_wait(k + 2, (u + 2) % 4)
                gat_start(k + 2, b, (u + 2) % 4)

    plsc.subcore_barrier()

    pltpu.sync_copy(acc.at[pl.ds(base, ROWS_PT)],
                    agg_o.at[c, pl.ds(base, ROWS_PT)])

    if deg:
        @pl.when(c == 0)
        def _():
            pltpu.sync_copy(dacc.at[pl.ds(base, ROWS_PT)],
                            deg_o.at[pl.ds(base, ROWS_PT)])


def _make_sc_segsum(deg):
    mesh = plsc.VectorSubcoreMesh(core_axis_name="c", subcore_axis_name="s")
    return pl.kernel(
        functools.partial(_sc_segsum_body, deg),
        mesh=mesh,
        out_type=[
            jax.ShapeDtypeStruct((NSC, NROWS, DH), jnp.float32),
            jax.ShapeDtypeStruct((NROWS,), jnp.float32),
        ],
        scratch_types=[
            pltpu.VMEM((4, 2, CHUNK), jnp.int32),
            pltpu.VMEM((CHUNK, DH), jnp.float32),
            pltpu.VMEM((CHUNK, DH), jnp.float32),
            pltpu.VMEM((CHUNK,), jnp.float32),
            pltpu.VMEM((ROWS_PT,), jnp.float32),
            pltpu.VMEM_SHARED((NROWS, DH), jnp.float32),
            pltpu.VMEM_SHARED((NROWS,), jnp.float32),
            pltpu.SemaphoreType.DMA,
            pltpu.SemaphoreType.DMA,
            pltpu.SemaphoreType.DMA,
            pltpu.SemaphoreType.DMA,
            pltpu.SemaphoreType.DMA,
            pltpu.SemaphoreType.DMA,
        ],
    )


@jax.jit
def _sc_segsum_deg(p2, idxp):
    return _make_sc_segsum(True)(p2, idxp)


@jax.jit
def _sc_segsum(p2, idxp):
    return _make_sc_segsum(False)(p2, idxp)


def _last_body(agg_ref, r_ref, deg_ref, o_ref):
    dinv = 1.0 / jnp.maximum(deg_ref[...], 1.0)
    o_ref[...] = agg_ref[0] * dinv + r_ref[...]


@jax.jit
def _tc_combine_last(agg, r, deg):
    return pl.pallas_call(
        _last_body,
        grid=(NBLK, NSC),
        in_specs=[
            pl.BlockSpec((1, NB, DH), lambda i, c: (c, i, 0)),
            pl.BlockSpec((NB, DH), lambda i, c: (i, c)),
            pl.BlockSpec((NB, 1), lambda i, c: (i, 0)),
        ],
        out_specs=pl.BlockSpec((NB, DH), lambda i, c: (i, c)),
        out_shape=jax.ShapeDtypeStruct((N, D), jnp.float32),
    )(agg, r, deg)


def kernel(x, edge_index, W_l, b_l, W_r, bn_gamma, bn_beta, bn_mean, bn_var):
    pad = EPAD - E
    iota = jnp.arange(pad, dtype=jnp.int32)
    src = jnp.concatenate([edge_index[0], iota % N])
    srcp = jnp.stack([src, src + N]).reshape(NSC, NTILE * NCHUNKS, 1, CHUNK)
    dstp = jnp.concatenate([edge_index[1], N + iota % (NROWS - N)])
    dstp = jnp.broadcast_to(dstp.reshape(1, NTILE * NCHUNKS, 1, CHUNK),
                            (NSC, NTILE * NCHUNKS, 1, CHUNK))
    idxp = jnp.concatenate([srcp, dstp], axis=2)

    wl_r = W_l.reshape(NUM_LAYERS, D, NSC, DH).transpose(0, 2, 1, 3)
    wr_r = W_r.reshape(NUM_LAYERS, D, NSC, DH).transpose(0, 2, 1, 3)
    wcat = jnp.concatenate([wl_r, wr_r], axis=3)

    p2, r = _tc_proj(x, wcat[0], b_l[0].reshape(1, D))
    agg, degp = _sc_segsum_deg(p2, idxp)
    deg_col = degp[:N].reshape(N, 1)
    for i in range(1, NUM_LAYERS):
        p2, r = _tc_fused(agg, r, deg_col,
                          bn_gamma[i - 1].reshape(1, D), bn_beta[i - 1].reshape(1, D),
                          bn_mean[i - 1].reshape(1, D), bn_var[i - 1].reshape(1, D),
                          wcat[i], b_l[i].reshape(1, D))
        agg, _ = _sc_segsum(p2, idxp)
    return _tc_combine_last(agg, r, deg_col)


# speedup vs baseline: 7.8995x; 1.0043x over previous
import functools

import jax
import jax.numpy as jnp
from jax import lax
from jax.experimental import pallas as pl
from jax.experimental.pallas import tpu as pltpu
from jax.experimental.pallas import tpu_sc as plsc

N = 10000
E = 160000
D = 256
DH = 128
NUM_LAYERS = 3
BN_EPS = 1e-5

NSC = 2
NTILE = 16
CHUNK = 128
NROWS = 10240
ROWS_PT = NROWS // NTILE
NCHUNKS = 80
EP_TILE = NCHUNKS * CHUNK
EPAD = NTILE * EP_TILE
NB = 2000
NBLK = N // NB


def _proj_body(h_ref, w_ref, b_ref, p_ref, r_ref):
    h = h_ref[...].astype(jnp.bfloat16)
    out = jnp.dot(h, w_ref[0].astype(jnp.bfloat16),
                  preferred_element_type=jnp.float32)
    p_ref[...] = out[:, :DH]
    r_ref[...] = out[:, DH:] + b_ref[...]


@jax.jit
def _tc_proj(h, wcat, b):
    return pl.pallas_call(
        _proj_body,
        grid=(NBLK, NSC),
        in_specs=[
            pl.BlockSpec((NB, D), lambda i, c: (i, 0)),
            pl.BlockSpec((1, D, D), lambda i, c: (c, 0, 0)),
            pl.BlockSpec((1, DH), lambda i, c: (0, c)),
        ],
        out_specs=[
            pl.BlockSpec((NB, DH), lambda i, c: (c * NBLK + i, 0)),
            pl.BlockSpec((NB, DH), lambda i, c: (i, c)),
        ],
        out_shape=[
            jax.ShapeDtypeStruct((NSC * N, DH), jnp.float32),
            jax.ShapeDtypeStruct((N, D), jnp.float32),
        ],
    )(h, wcat, b)


def _fused_body(agg_ref, r_ref, deg_ref, g_ref, be_ref, m_ref, v_ref,
                w_ref, b_ref, p_ref, r2_ref):
    dinv = 1.0 / jnp.maximum(deg_ref[...], 1.0)
    h = agg_ref[...].reshape(NB, D) * dinv + r_ref[...]
    scale = g_ref[...] * lax.rsqrt(v_ref[...] + BN_EPS)
    h = jnp.maximum((h - m_ref[...]) * scale + be_ref[...], 0.0)
    out = jnp.dot(h.astype(jnp.bfloat16), w_ref[0].astype(jnp.bfloat16),
                  preferred_element_type=jnp.float32)
    p_ref[...] = out[:, :DH]
    r2_ref[...] = out[:, DH:] + b_ref[...]


@jax.jit
def _tc_fused(agg, r, deg, g, be, m, v, wcat, b):
    return pl.pallas_call(
        _fused_body,
        grid=(NBLK, NSC),
        in_specs=[
            pl.BlockSpec((NB, NSC, DH), lambda i, c: (i, 0, 0)),
            pl.BlockSpec((NB, D), lambda i, c: (i, 0)),
            pl.BlockSpec((NB, 1), lambda i, c: (i, 0)),
            pl.BlockSpec((1, D), lambda i, c: (0, 0)),
            pl.BlockSpec((1, D), lambda i, c: (0, 0)),
            pl.BlockSpec((1, D), lambda i, c: (0, 0)),
            pl.BlockSpec((1, D), lambda i, c: (0, 0)),
            pl.BlockSpec((1, D, D), lambda i, c: (c, 0, 0)),
            pl.BlockSpec((1, DH), lambda i, c: (0, c)),
        ],
        out_specs=[
            pl.BlockSpec((NB, DH), lambda i, c: (c * NBLK + i, 0)),
            pl.BlockSpec((NB, DH), lambda i, c: (i, c)),
        ],
        out_shape=[
            jax.ShapeDtypeStruct((NSC * N, DH), jnp.float32),
            jax.ShapeDtypeStruct((N, D), jnp.float32),
        ],
    )(agg, r, deg, g, be, m, v, wcat, b)


def _sc_segsum_body(deg, p2, idxh, agg_o, deg_o,
                    idxv, rows0, rows1, ones, z1,
                    acc, dacc,
                    semi0, semi1, semi2, semi3, semg0, semg1):
    c = lax.axis_index("c")
    s = lax.axis_index("s")
    rows = (rows0, rows1)
    semi = (semi0, semi1, semi2, semi3)
    semg = (semg0, semg1)
    row0 = s * NCHUNKS

    def idx_start(k, q):
        pltpu.async_copy(idxh.at[c, row0 + k], idxv.at[q], semi[q])

    def idx_wait(k, q):
        pltpu.make_async_copy(idxh.at[c, row0 + k], idxv.at[q], semi[q]).wait()

    def gat_start(k, b, q):
        pltpu.async_copy(p2.at[idxv.at[q, 0]], rows[b], semg[b])

    def gat_wait(b):
        pltpu.make_async_copy(p2.at[idxv.at[0, 0]], rows[b], semg[b]).wait()

    for q in range(4):
        idx_start(q, q)

    if deg:
        @pl.loop(0, CHUNK, step=16)
        def _(j):
            ones[pl.ds(j, 16)] = jnp.full((16,), 1.0, jnp.float32)

        @pl.loop(0, ROWS_PT, step=16)
        def _(j):
            z1[pl.ds(j, 16)] = jnp.zeros((16,), jnp.float32)

    @pl.loop(0, CHUNK)
    def _(r):
        @pl.loop(0, DH, step=16)
        def _(j):
            rows0[r, pl.ds(j, 16)] = jnp.zeros((16,), jnp.float32)

    base = s * ROWS_PT
    for k in range(ROWS_PT // CHUNK):
        pltpu.sync_copy(rows0, acc.at[pl.ds(base + k * CHUNK, CHUNK)])
    if deg:
        pltpu.sync_copy(z1, dacc.at[pl.ds(base, ROWS_PT)])
    plsc.subcore_barrier()

    idx_wait(0, 0)
    gat_start(0, 0, 0)
    idx_wait(1, 1)
    gat_start(1, 1, 1)

    @pl.loop(0, NCHUNKS, step=4)
    def _(i):
        for u in range(4):
            k = i + u
            b = u % 2
            q = u
            gat_wait(b)
            pltpu.sync_copy(rows[b], acc.at[idxv.at[q, 1]], add=True)
            if deg:
                @pl.when(c == 0)
                def _():
                    pltpu.sync_copy(ones, dacc.at[idxv.at[q, 1]], add=True)

            @pl.when(k + 4 < NCHUNKS)
            def _():
                idx_start(k + 4, q)

            @pl.when(k + 2 < NCHUNKS)
            def _():
                idx_wait(k + 2, (u + 2) % 4)
                gat_start(k + 2, b, (u + 2) % 4)

    plsc.subcore_barrier()

    pltpu.sync_copy(acc.at[pl.ds(base, ROWS_PT)],
                    agg_o.at[pl.ds(base, ROWS_PT), c])

    if deg:
        @pl.when(c == 0)
        def _():
            pltpu.sync_copy(dacc.at[pl.ds(base, ROWS_PT)],
                            deg_o.at[pl.ds(base, ROWS_PT)])


def _make_sc_segsum(deg):
    mesh = plsc.VectorSubcoreMesh(core_axis_name="c", subcore_axis_name="s")
    return pl.kernel(
        functools.partial(_sc_segsum_body, deg),
        mesh=mesh,
        out_type=[
            jax.ShapeDtypeStruct((NROWS, NSC, DH), jnp.float32),
            jax.ShapeDtypeStruct((NROWS,), jnp.float32),
        ],
        scratch_types=[
            pltpu.VMEM((4, 2, CHUNK), jnp.int32),
            pltpu.VMEM((CHUNK, DH), jnp.float32),
            pltpu.VMEM((CHUNK, DH), jnp.float32),
            pltpu.VMEM((CHUNK,), jnp.float32),
            pltpu.VMEM((ROWS_PT,), jnp.float32),
            pltpu.VMEM_SHARED((NROWS, DH), jnp.float32),
            pltpu.VMEM_SHARED((NROWS,), jnp.float32),
            pltpu.SemaphoreType.DMA,
            pltpu.SemaphoreType.DMA,
            pltpu.SemaphoreType.DMA,
            pltpu.SemaphoreType.DMA,
            pltpu.SemaphoreType.DMA,
            pltpu.SemaphoreType.DMA,
        ],
    )


@jax.jit
def _sc_segsum_deg(p2, idxp):
    return _make_sc_segsum(True)(p2, idxp)


@jax.jit
def _sc_segsum(p2, idxp):
    return _make_sc_segsum(False)(p2, idxp)


def _last_body(agg_ref, r_ref, deg_ref, o_ref):
    dinv = 1.0 / jnp.maximum(deg_ref[...], 1.0)
    o_ref[...] = agg_ref[...].reshape(NB, D) * dinv + r_ref[...]


@jax.jit
def _tc_combine_last(agg, r, deg):
    return pl.pallas_call(
        _last_body,
        grid=(NBLK,),
        in_specs=[
            pl.BlockSpec((NB, NSC, DH), lambda i: (i, 0, 0)),
            pl.BlockSpec((NB, D), lambda i: (i, 0)),
            pl.BlockSpec((NB, 1), lambda i: (i, 0)),
        ],
        out_specs=pl.BlockSpec((NB, D), lambda i: (i, 0)),
        out_shape=jax.ShapeDtypeStruct((N, D), jnp.float32),
    )(agg, r, deg)


def kernel(x, edge_index, W_l, b_l, W_r, bn_gamma, bn_beta, bn_mean, bn_var):
    pad = EPAD - E
    iota = jnp.arange(pad, dtype=jnp.int32)
    src = jnp.concatenate([edge_index[0], iota % N])
    srcp = jnp.stack([src, src + N]).reshape(NSC, NTILE * NCHUNKS, 1, CHUNK)
    dstp = jnp.concatenate([edge_index[1], N + iota % (NROWS - N)])
    dstp = jnp.broadcast_to(dstp.reshape(1, NTILE * NCHUNKS, 1, CHUNK),
                            (NSC, NTILE * NCHUNKS, 1, CHUNK))
    idxp = jnp.concatenate([srcp, dstp], axis=2)

    wl_r = W_l.reshape(NUM_LAYERS, D, NSC, DH).transpose(0, 2, 1, 3)
    wr_r = W_r.reshape(NUM_LAYERS, D, NSC, DH).transpose(0, 2, 1, 3)
    wcat = jnp.concatenate([wl_r, wr_r], axis=3)

    p2, r = _tc_proj(x, wcat[0], b_l[0].reshape(1, D))
    agg, degp = _sc_segsum_deg(p2, idxp)
    deg_col = degp[:N].reshape(N, 1)
    for i in range(1, NUM_LAYERS):
        p2, r = _tc_fused(agg, r, deg_col,
                          bn_gamma[i - 1].reshape(1, D), bn_beta[i - 1].reshape(1, D),
                          bn_mean[i - 1].reshape(1, D), bn_var[i - 1].reshape(1, D),
                          wcat[i], b_l[i].reshape(1, D))
        agg, _ = _sc_segsum(p2, idxp)
    return _tc_combine_last(agg, r, deg_col)
